# baseline (speedup 1.0000x reference)
import functools

import jax
import jax.numpy as jnp
from jax import lax
from jax.experimental import pallas as pl
from jax.experimental.pallas import tpu as pltpu
from jax.experimental.pallas import tpu_sc as plsc

B, H, W = 4, 32, 64
CHUNK = 4
NQ = W // CHUNK
NPAD = 48
MAX_VALUE = 50.0
NR = NQ * H * B
NWORK = 32
RPW = NR // NWORK

_DN = (((1,), (1,)), ((), ()))
_DNK = (((1,), (0,)), ((), ()))


@functools.partial(
    pl.kernel,
    out_type=jax.ShapeDtypeStruct((NWORK, 16, RPW), jnp.float32),
    mesh=plsc.VectorSubcoreMesh(core_axis_name="c", subcore_axis_name="s"),
    scratch_types=[pltpu.VMEM((2 * CHUNK, NR), jnp.float32),
                   pltpu.VMEM((16, RPW), jnp.float32)],
)
def _pattern_sc(cols_hbm, out_hbm, cols_v, outb_v):
    wid = lax.axis_index("s") * 2 + lax.axis_index("c")
    pltpu.sync_copy(cols_hbm, cols_v)
    for j in range(RPW // 16):
        col0 = wid * RPW + j * 16
        patt = jnp.zeros((16,), jnp.int32)
        for k in range(CHUNK):
            g = cols_v[k, pl.ds(col0, 16)]
            m = cols_v[k + CHUNK, pl.ds(col0, 16)]
            bit = jnp.logical_and(g > 0, m > 0)
            patt = patt + jnp.where(bit, 1 << k, 0)
        for p in range(16):
            outb_v[p, pl.ds(j * 16, 16)] = jnp.where(patt == p, 1.0, 0.0)
    pltpu.sync_copy(outb_v, out_hbm.at[wid])


def _lv_kernel(e0_ref, e0b_ref, e1_ref, e1b_ref, c0_ref, c0b_ref,
               c1_ref, c1b_ref, v_ref, a0b_ref, lv_ref):
    f32 = jnp.float32
    ri = jax.lax.broadcasted_iota(jnp.int32, (16 * CHUNK, 1), 0)
    bitcol = (jax.lax.shift_right_logical(ri // CHUNK, ri % CHUNK) & 1
              ).astype(f32)
    poscol = (ri % CHUNK).astype(f32) / CHUNK
    x = jnp.concatenate([bitcol, poscol], axis=1)
    h1 = jax.nn.relu(jax.lax.dot_general(x, e0_ref[...], _DN,
                                         preferred_element_type=f32)
                     + e0b_ref[...])
    h2 = jax.nn.relu(jax.lax.dot_general(h1, e1_ref[...], _DN,
                                         preferred_element_type=f32)
                     + e1b_ref[...])
    pi = jax.lax.broadcasted_iota(jnp.int32, (16, 16 * CHUNK), 0)
    rj = jax.lax.broadcasted_iota(jnp.int32, (16, 16 * CHUNK), 1)
    mmat = jnp.where((rj // CHUNK) == pi, 0.25, 0.0).astype(f32)
    pooled = jax.lax.dot_general(mmat, h2, _DNK,
                                 preferred_element_type=f32)
    cc = jax.nn.relu(jax.lax.dot_general(pooled, c0_ref[...], _DN,
                                         preferred_element_type=f32)
                     + c0b_ref[...])
    c2 = jax.nn.relu(jnp.sum(cc * c1_ref[...], axis=1, keepdims=True)
                     + c1b_ref[0, 0])
    lut = jnp.round(c2)
    lv_ref[...] = jax.lax.dot_general(lut, v_ref[...], _DNK,
                                      preferred_element_type=f32) + a0b_ref[...]


def _scan_kernel(onehot_ref, lv_ref, u_ref, a1_ref,
                 a1b_ref, a2r_ref, a2b_ref, out_ref, cv_ref, e_ref):
    f32 = jnp.float32
    u = u_ref[...]
    a1 = a1_ref[...]
    a1b = a1b_ref[...]
    a2r = a2r_ref[...]
    a2b = a2b_ref[0, 0]

    cv_ref[...] = jax.lax.dot_general(onehot_ref[...], lv_ref[...], _DNK,
                                      preferred_element_type=f32)
    cv0 = lv_ref[0:1, :]

    def mlp(R, cv):
        h1 = jax.nn.relu(R * u + cv)
        a = jax.nn.relu(jax.lax.dot_general(h1, a1, _DNK,
                                            preferred_element_type=f32) + a1b)
        Ob = jax.lax.dot_general(a, a2r, _DNK, preferred_element_type=f32)
        return jnp.round((Ob + a2b) * MAX_VALUE)

    def padded_run(R):
        def cond(c):
            return jnp.logical_and(c[0] < NPAD, c[2])

        def body(c):
            i, rr, _ = c
            r2 = mlp(rr, cv0)
            return i + 1, r2, jnp.any(r2 != rr)

        return jax.lax.while_loop(cond, body, (jnp.int32(0), R, True))[1]

    rstar = padded_run(jnp.zeros((1, 128), dtype=f32))

    sub = jax.lax.broadcasted_iota(jnp.int32, (H * B, 128), 0)
    R = jnp.where(sub < B, 0.0, rstar)
    for q in range(NQ):
        R = mlp(R, cv_ref[pl.ds(q * H * B, H * B), :])
    e_ref[...] = padded_run(R)

    def seq_row(h, st):
        for q in range(NQ):
            st = mlp(st, cv_ref[pl.ds(q * H * B + h * B, B), :])
        return padded_run(st)

    def stitch(h, st):
        eh = e_ref[pl.ds(h * B, B), :]
        return jax.lax.cond(jnp.all(st == rstar),
                            lambda s: eh, lambda s: seq_row(h, s), st)

    state = jax.lax.fori_loop(1, H, stitch, e_ref[0:B, :])
    out_ref[...] = state[:, 0:1]


def kernel(grid, mask, sub_enc_w0, sub_enc_b0, sub_enc_w1, sub_enc_b1,
           sub_cls_w0, sub_cls_b0, sub_cls_w1, sub_cls_b1,
           add_w0, add_b0, add_w1, add_b1, add_w2, add_b2):
    f32 = jnp.float32
    gc = jnp.transpose(grid.reshape(B, H, NQ, CHUNK), (3, 2, 1, 0)
                       ).reshape(CHUNK, NR)
    mc = jnp.transpose(mask.reshape(B, H, NQ, CHUNK), (3, 2, 1, 0)
                       ).reshape(CHUNK, NR)
    oh3 = _pattern_sc(jnp.concatenate([gc, mc], axis=0))
    onehot = jnp.transpose(oh3, (0, 2, 1)).reshape(NR, 16)
    lv = pl.pallas_call(
        _lv_kernel,
        out_shape=jax.ShapeDtypeStruct((16, 128), f32),
    )(sub_enc_w0, sub_enc_b0.reshape(1, 64),
      sub_enc_w1, sub_enc_b1.reshape(1, 64),
      sub_cls_w0, sub_cls_b0.reshape(1, 32),
      sub_cls_w1, sub_cls_b1.reshape(1, 1),
      (add_w0[:, 1] / MAX_VALUE).reshape(1, 128),
      add_b0.reshape(1, 128))
    total = pl.pallas_call(
        _scan_kernel,
        out_shape=jax.ShapeDtypeStruct((B, 1), f32),
        scratch_shapes=[pltpu.VMEM((NR, 128), f32),
                        pltpu.VMEM((H * B, 128), f32)],
    )(onehot, lv,
      (add_w0[:, 0] / MAX_VALUE).reshape(1, 128),
      add_w1.T, add_b1.reshape(1, 128),
      jnp.broadcast_to(add_w2.reshape(128, 1), (128, 128)),
      add_b2.reshape(1, 1))
    return total.reshape(B)


# speedup vs baseline: 1.1308x; 1.1308x over previous
import functools

import jax
import jax.numpy as jnp
from jax import lax
from jax.experimental import pallas as pl
from jax.experimental.pallas import tpu as pltpu
from jax.experimental.pallas import tpu_sc as plsc

B, H, W = 4, 32, 64
CHUNK = 4
NQ = W // CHUNK
NPAD = 48
MAX_VALUE = 50.0
NR = NQ * H * B
NWORK = 32
RPW = NR // NWORK

_DN = (((1,), (1,)), ((), ()))
_DNK = (((1,), (0,)), ((), ()))


@functools.partial(
    pl.kernel,
    out_type=jax.ShapeDtypeStruct((16, NR), jnp.float32),
    mesh=plsc.VectorSubcoreMesh(core_axis_name="c", subcore_axis_name="s"),
    scratch_types=[pltpu.VMEM((2 * CHUNK, 128), jnp.float32),
                   pltpu.VMEM((16, 128), jnp.float32)],
)
def _pattern_sc(cols_hbm, out_hbm, cols_v, outb_v):
    sid = lax.axis_index("s")

    @pl.when(lax.axis_index("c") == 0)
    def _():
        pltpu.sync_copy(cols_hbm.at[:, pl.ds(sid * 128, 128)], cols_v)
        for j in range(128 // 16):
            col0 = j * 16
            patt = jnp.zeros((16,), jnp.int32)
            for k in range(CHUNK):
                g = cols_v[k, pl.ds(col0, 16)]
                m = cols_v[k + CHUNK, pl.ds(col0, 16)]
                bit = jnp.logical_and(g > 0, m > 0)
                patt = patt + jnp.where(bit, 1 << k, 0)
            for p in range(16):
                outb_v[p, pl.ds(col0, 16)] = jnp.where(patt == p, 1.0, 0.0)
        pltpu.sync_copy(outb_v, out_hbm.at[:, pl.ds(sid * 128, 128)])


def _lv_kernel(e0_ref, e0b_ref, e1_ref, e1b_ref, c0_ref, c0b_ref,
               c1_ref, c1b_ref, v_ref, a0b_ref, lv_ref):
    f32 = jnp.float32
    ri = jax.lax.broadcasted_iota(jnp.int32, (16 * CHUNK, 1), 0)
    bitcol = (jax.lax.shift_right_logical(ri // CHUNK, ri % CHUNK) & 1
              ).astype(f32)
    poscol = (ri % CHUNK).astype(f32) / CHUNK
    x = jnp.concatenate([bitcol, poscol], axis=1)
    h1 = jax.nn.relu(jax.lax.dot_general(x, e0_ref[...], _DN,
                                         preferred_element_type=f32)
                     + e0b_ref[...])
    h2 = jax.nn.relu(jax.lax.dot_general(h1, e1_ref[...], _DN,
                                         preferred_element_type=f32)
                     + e1b_ref[...])
    pi = jax.lax.broadcasted_iota(jnp.int32, (16, 16 * CHUNK), 0)
    rj = jax.lax.broadcasted_iota(jnp.int32, (16, 16 * CHUNK), 1)
    mmat = jnp.where((rj // CHUNK) == pi, 0.25, 0.0).astype(f32)
    pooled = jax.lax.dot_general(mmat, h2, _DNK,
                                 preferred_element_type=f32)
    cc = jax.nn.relu(jax.lax.dot_general(pooled, c0_ref[...], _DN,
                                         preferred_element_type=f32)
                     + c0b_ref[...])
    c2 = jax.nn.relu(jnp.sum(cc * c1_ref[...], axis=1, keepdims=True)
                     + c1b_ref[0, 0])
    lut = jnp.round(c2)
    lv_ref[...] = jax.lax.dot_general(lut, v_ref[...], _DNK,
                                      preferred_element_type=f32) + a0b_ref[...]


def _scan_kernel(onehot_ref, lv_ref, u_ref, a1_ref,
                 a1b_ref, a2r_ref, a2b_ref, out_ref, cv_ref, e_ref):
    f32 = jnp.float32
    u = u_ref[...]
    a1 = a1_ref[...]
    a1b = a1b_ref[...]
    a2r = a2r_ref[...]
    a2b = a2b_ref[0, 0]

    cv_ref[...] = jax.lax.dot_general(onehot_ref[...], lv_ref[...],
                                      (((0,), (0,)), ((), ())),
                                      preferred_element_type=f32)
    cv0 = lv_ref[0:1, :]

    def mlp(R, cv):
        h1 = jax.nn.relu(R * u + cv)
        a = jax.nn.relu(jax.lax.dot_general(h1, a1, _DNK,
                                            preferred_element_type=f32) + a1b)
        Ob = jax.lax.dot_general(a, a2r, _DNK, preferred_element_type=f32)
        return jnp.round((Ob + a2b) * MAX_VALUE)

    def padded_run(R):
        def cond(c):
            return jnp.logical_and(c[0] < NPAD, c[2])

        def body(c):
            i, rr, _ = c
            r2 = mlp(rr, cv0)
            return i + 1, r2, jnp.any(r2 != rr)

        return jax.lax.while_loop(cond, body, (jnp.int32(0), R, True))[1]

    rstar = padded_run(jnp.zeros((1, 128), dtype=f32))

    sub = jax.lax.broadcasted_iota(jnp.int32, (H * B, 128), 0)
    R = jnp.where(sub < B, 0.0, rstar)
    for q in range(NQ):
        R = mlp(R, cv_ref[pl.ds(q * H * B, H * B), :])
    e_ref[...] = padded_run(R)

    def seq_row(h, st):
        for q in range(NQ):
            st = mlp(st, cv_ref[pl.ds(q * H * B + h * B, B), :])
        return padded_run(st)

    def stitch(h, st):
        eh = e_ref[pl.ds(h * B, B), :]
        return jax.lax.cond(jnp.all(st == rstar),
                            lambda s: eh, lambda s: seq_row(h, s), st)

    state = jax.lax.fori_loop(1, H, stitch, e_ref[0:B, :])
    out_ref[...] = state[:, 0:1]


def kernel(grid, mask, sub_enc_w0, sub_enc_b0, sub_enc_w1, sub_enc_b1,
           sub_cls_w0, sub_cls_b0, sub_cls_w1, sub_cls_b1,
           add_w0, add_b0, add_w1, add_b1, add_w2, add_b2):
    f32 = jnp.float32
    gc = jnp.transpose(grid.reshape(B, H, NQ, CHUNK), (3, 2, 1, 0)
                       ).reshape(CHUNK, NR)
    mc = jnp.transpose(mask.reshape(B, H, NQ, CHUNK), (3, 2, 1, 0)
                       ).reshape(CHUNK, NR)
    onehot = _pattern_sc(jnp.concatenate([gc, mc], axis=0))
    lv = pl.pallas_call(
        _lv_kernel,
        out_shape=jax.ShapeDtypeStruct((16, 128), f32),
    )(sub_enc_w0, sub_enc_b0.reshape(1, 64),
      sub_enc_w1, sub_enc_b1.reshape(1, 64),
      sub_cls_w0, sub_cls_b0.reshape(1, 32),
      sub_cls_w1, sub_cls_b1.reshape(1, 1),
      (add_w0[:, 1] / MAX_VALUE).reshape(1, 128),
      add_b0.reshape(1, 128))
    total = pl.pallas_call(
        _scan_kernel,
        out_shape=jax.ShapeDtypeStruct((B, 1), f32),
        scratch_shapes=[pltpu.VMEM((NR, 128), f32),
                        pltpu.VMEM((H * B, 128), f32)],
    )(onehot, lv,
      (add_w0[:, 0] / MAX_VALUE).reshape(1, 128),
      add_w1.T, add_b1.reshape(1, 128),
      jnp.broadcast_to(add_w2.reshape(128, 1), (128, 128)),
      add_b2.reshape(1, 1))
    return total.reshape(B)


# speedup vs baseline: 1.1324x; 1.0014x over previous
import functools

import jax
import jax.numpy as jnp
from jax import lax
from jax.experimental import pallas as pl
from jax.experimental.pallas import tpu as pltpu
from jax.experimental.pallas import tpu_sc as plsc

B, H, W = 4, 32, 64
CHUNK = 4
NQ = W // CHUNK
NPAD = 48
MAX_VALUE = 50.0
NR = NQ * H * B
NWORK = 32
RPW = NR // NWORK

_DN = (((1,), (1,)), ((), ()))
_DNK = (((1,), (0,)), ((), ()))


@functools.partial(
    pl.kernel,
    out_type=jax.ShapeDtypeStruct((16, NR), jnp.float32),
    mesh=plsc.VectorSubcoreMesh(core_axis_name="c", subcore_axis_name="s"),
    scratch_types=[pltpu.VMEM((2 * CHUNK, 128), jnp.float32),
                   pltpu.VMEM((16, 128), jnp.float32)],
)
def _pattern_sc(cols_hbm, out_hbm, cols_v, outb_v):
    sid = lax.axis_index("s")

    @pl.when(lax.axis_index("c") == 0)
    def _():
        pltpu.sync_copy(cols_hbm.at[:, pl.ds(sid * 128, 128)], cols_v)
        for j in range(128 // 16):
            col0 = j * 16
            patt = jnp.zeros((16,), jnp.int32)
            for k in range(CHUNK):
                g = cols_v[k, pl.ds(col0, 16)]
                m = cols_v[k + CHUNK, pl.ds(col0, 16)]
                bit = jnp.logical_and(g > 0, m > 0)
                patt = patt + jnp.where(bit, 1 << k, 0)
            for p in range(16):
                outb_v[p, pl.ds(col0, 16)] = jnp.where(patt == p, 1.0, 0.0)
        pltpu.sync_copy(outb_v, out_hbm.at[:, pl.ds(sid * 128, 128)])


def _lv_kernel(e0_ref, e0b_ref, e1_ref, e1b_ref, c0_ref, c0b_ref,
               c1_ref, c1b_ref, v_ref, a0b_ref, u_ref, a1_ref, a1b_ref,
               a2r_ref, a2b_ref, lv_ref, rstar_ref):
    f32 = jnp.float32
    ri = jax.lax.broadcasted_iota(jnp.int32, (16 * CHUNK, 1), 0)
    bitcol = (jax.lax.shift_right_logical(ri // CHUNK, ri % CHUNK) & 1
              ).astype(f32)
    poscol = (ri % CHUNK).astype(f32) / CHUNK
    x = jnp.concatenate([bitcol, poscol], axis=1)
    h1 = jax.nn.relu(jax.lax.dot_general(x, e0_ref[...], _DN,
                                         preferred_element_type=f32)
                     + e0b_ref[...])
    h2 = jax.nn.relu(jax.lax.dot_general(h1, e1_ref[...], _DN,
                                         preferred_element_type=f32)
                     + e1b_ref[...])
    pi = jax.lax.broadcasted_iota(jnp.int32, (16, 16 * CHUNK), 0)
    rj = jax.lax.broadcasted_iota(jnp.int32, (16, 16 * CHUNK), 1)
    mmat = jnp.where((rj // CHUNK) == pi, 0.25, 0.0).astype(f32)
    pooled = jax.lax.dot_general(mmat, h2, _DNK,
                                 preferred_element_type=f32)
    cc = jax.nn.relu(jax.lax.dot_general(pooled, c0_ref[...], _DN,
                                         preferred_element_type=f32)
                     + c0b_ref[...])
    c2 = jax.nn.relu(jnp.sum(cc * c1_ref[...], axis=1, keepdims=True)
                     + c1b_ref[0, 0])
    lut = jnp.round(c2)
    lv = jax.lax.dot_general(lut, v_ref[...], _DNK,
                             preferred_element_type=f32) + a0b_ref[...]
    lv_ref[...] = lv

    u = u_ref[...]
    a1 = a1_ref[...]
    a1b = a1b_ref[...]
    a2r = a2r_ref[...]
    a2b = a2b_ref[0, 0]
    cv0 = lv[0:1, :]

    def mlp(R, cv):
        h1 = jax.nn.relu(R * u + cv)
        a = jax.nn.relu(jax.lax.dot_general(h1, a1, _DNK,
                                            preferred_element_type=f32) + a1b)
        Ob = jax.lax.dot_general(a, a2r, _DNK, preferred_element_type=f32)
        return jnp.round((Ob + a2b) * MAX_VALUE)

    def cond(c):
        return jnp.logical_and(c[0] < NPAD, c[2])

    def body(c):
        i, rr, _ = c
        r2 = mlp(rr, cv0)
        return i + 1, r2, jnp.any(r2 != rr)

    rstar_ref[...] = jax.lax.while_loop(
        cond, body, (jnp.int32(0), jnp.zeros((1, 128), f32), True))[1]


def _scan_kernel(onehot_ref, lv_ref, rstar_in_ref, u_ref, a1_ref,
                 a1b_ref, a2r_ref, a2b_ref, out_ref, cv_ref, e_ref):
    f32 = jnp.float32
    u = u_ref[...]
    a1 = a1_ref[...]
    a1b = a1b_ref[...]
    a2r = a2r_ref[...]
    a2b = a2b_ref[0, 0]

    cv_ref[...] = jax.lax.dot_general(onehot_ref[...], lv_ref[...],
                                      (((0,), (0,)), ((), ())),
                                      preferred_element_type=f32)
    cv0 = lv_ref[0:1, :]

    def mlp(R, cv):
        h1 = jax.nn.relu(R * u + cv)
        a = jax.nn.relu(jax.lax.dot_general(h1, a1, _DNK,
                                            preferred_element_type=f32) + a1b)
        Ob = jax.lax.dot_general(a, a2r, _DNK, preferred_element_type=f32)
        return jnp.round((Ob + a2b) * MAX_VALUE)

    def padded_run(R):
        def cond(c):
            return jnp.logical_and(c[0] < NPAD, c[2])

        def body(c):
            i, rr, _ = c
            r2 = mlp(rr, cv0)
            return i + 1, r2, jnp.any(r2 != rr)

        return jax.lax.while_loop(cond, body, (jnp.int32(0), R, True))[1]

    rstar = rstar_in_ref[...]

    sub = jax.lax.broadcasted_iota(jnp.int32, (H * B, 128), 0)
    R = jnp.where(sub < B, 0.0, rstar)
    for q in range(NQ):
        R = mlp(R, cv_ref[pl.ds(q * H * B, H * B), :])
    e_ref[...] = padded_run(R)

    def seq_row(h, st):
        for q in range(NQ):
            st = mlp(st, cv_ref[pl.ds(q * H * B + h * B, B), :])
        return padded_run(st)

    def stitch(h, st):
        eh = e_ref[pl.ds(h * B, B), :]
        return jax.lax.cond(jnp.all(st == rstar),
                            lambda s: eh, lambda s: seq_row(h, s), st)

    state = jax.lax.fori_loop(1, H, stitch, e_ref[0:B, :])
    out_ref[...] = state[:, 0:1]


def kernel(grid, mask, sub_enc_w0, sub_enc_b0, sub_enc_w1, sub_enc_b1,
           sub_cls_w0, sub_cls_b0, sub_cls_w1, sub_cls_b1,
           add_w0, add_b0, add_w1, add_b1, add_w2, add_b2):
    f32 = jnp.float32
    gc = jnp.transpose(grid.reshape(B, H, NQ, CHUNK), (3, 2, 1, 0)
                       ).reshape(CHUNK, NR)
    mc = jnp.transpose(mask.reshape(B, H, NQ, CHUNK), (3, 2, 1, 0)
                       ).reshape(CHUNK, NR)
    onehot = _pattern_sc(jnp.concatenate([gc, mc], axis=0))
    u = (add_w0[:, 0] / MAX_VALUE).reshape(1, 128)
    a1t = add_w1.T
    a1b = add_b1.reshape(1, 128)
    a2r = jnp.broadcast_to(add_w2.reshape(128, 1), (128, 128))
    a2b = add_b2.reshape(1, 1)
    lv, rstar = pl.pallas_call(
        _lv_kernel,
        out_shape=[jax.ShapeDtypeStruct((16, 128), f32),
                   jax.ShapeDtypeStruct((1, 128), f32)],
    )(sub_enc_w0, sub_enc_b0.reshape(1, 64),
      sub_enc_w1, sub_enc_b1.reshape(1, 64),
      sub_cls_w0, sub_cls_b0.reshape(1, 32),
      sub_cls_w1, sub_cls_b1.reshape(1, 1),
      (add_w0[:, 1] / MAX_VALUE).reshape(1, 128),
      add_b0.reshape(1, 128), u, a1t, a1b, a2r, a2b)
    total = pl.pallas_call(
        _scan_kernel,
        out_shape=jax.ShapeDtypeStruct((B, 1), f32),
        scratch_shapes=[pltpu.VMEM((NR, 128), f32),
                        pltpu.VMEM((H * B, 128), f32)],
    )(onehot, lv, rstar, u, a1t, a1b, a2r, a2b)
    return total.reshape(B)
